# baseline (speedup 1.0000x reference)
import functools

import jax
import jax.numpy as jnp
from jax import lax
from jax.experimental import pallas as pl
from jax.experimental.pallas import tpu as pltpu
from jax.experimental.pallas import tpu_sc as plsc

EMBED_DIM = 32
PHYS_W = 128
FOLD = PHYS_W // EMBED_DIM

NUM_CORES = 2
NUM_SUBCORES = 16
NW = NUM_CORES * NUM_SUBCORES

C = 512
CT = C // 16


def _sc_gather(idx, table_phys, n_rows):
    rows_per_w = n_rows // NW
    n_chunks = rows_per_w // C

    mesh = plsc.VectorSubcoreMesh(core_axis_name="c", subcore_axis_name="s")

    @functools.partial(
        pl.kernel,
        mesh=mesh,
        out_type=jax.ShapeDtypeStruct((n_rows // FOLD, PHYS_W), jnp.float32),
        compiler_params=pltpu.CompilerParams(needs_layout_passes=False),
        scratch_types=[
            pltpu.VMEM((rows_per_w,), jnp.int32),
            pltpu.VMEM((rows_per_w,), jnp.int32),
            pltpu.VMEM((C, PHYS_W), jnp.float32),
            pltpu.VMEM((C // FOLD, PHYS_W), jnp.float32),
            pltpu.SemaphoreType.DMA,
        ],
    )
    def body(idx_hbm, table_hbm, out_hbm, idx_v, phys_v, gath, outb, sem):
        wid = lax.axis_index("s") * NUM_CORES + lax.axis_index("c")
        base = wid * rows_per_w
        pltpu.sync_copy(idx_hbm.at[pl.ds(base, rows_per_w)], idx_v)

        def shift_body(i, carry):
            v = idx_v[pl.ds(i * 16, 16)]
            phys_v[pl.ds(i * 16, 16)] = v >> 2
            return carry

        lax.fori_loop(0, rows_per_w // 16, shift_body, 0)

        iota = lax.iota(jnp.int32, 16)

        def chunk_body(c, carry):
            pltpu.async_copy(
                table_hbm.at[phys_v.at[pl.ds(c * C, C)]], gath, sem
            ).wait()

            def ext_body(t, carry2):
                j0 = t * 16
                lv = idx_v[pl.ds(c * C + j0, 16)]
                sub = lv & 3
                grow = j0 + iota
                gcol = sub * EMBED_DIM
                orow = t * 4 + (iota >> 2)
                ocol = (iota & 3) * EMBED_DIM
                for k in range(EMBED_DIM):
                    vals = plsc.load_gather(gath, [grow, gcol + k])
                    plsc.store_scatter(outb, [orow, ocol + k], vals)
                return carry2

            lax.fori_loop(0, CT, ext_body, 0)

            off = pl.multiple_of((base + c * C) // FOLD, 8)
            pltpu.sync_copy(outb, out_hbm.at[pl.ds(off, C // FOLD)])
            return carry

        lax.fori_loop(0, n_chunks, chunk_body, 0)

    return body(idx, table_phys)


def kernel(x, table):
    batch, n_fields = x.shape
    n_rows = batch * n_fields
    idx = x.reshape(n_rows).astype(jnp.int32)
    table_phys = table.reshape(table.shape[0] // FOLD, PHYS_W)
    out = _sc_gather(idx, table_phys, n_rows)
    return out.reshape(batch, n_fields, EMBED_DIM)


# speedup vs baseline: 1.4375x; 1.4375x over previous
import functools

import jax
import jax.numpy as jnp
from jax import lax
from jax.experimental import pallas as pl
from jax.experimental.pallas import tpu as pltpu
from jax.experimental.pallas import tpu_sc as plsc

EMBED_DIM = 32
PHYS_W = 128
FOLD = PHYS_W // EMBED_DIM

NUM_CORES = 2
NUM_SUBCORES = 16
NW = NUM_CORES * NUM_SUBCORES

BB = 512


def _sc_embed(xt, table_phys, batch, n_fields):
    mesh = plsc.VectorSubcoreMesh(core_axis_name="c", subcore_axis_name="s")
    n_idx = n_fields * BB

    @functools.partial(
        pl.kernel,
        mesh=mesh,
        out_type=jax.ShapeDtypeStruct((n_fields, EMBED_DIM, batch), jnp.float32),
        compiler_params=pltpu.CompilerParams(needs_layout_passes=False),
        scratch_types=[
            pltpu.VMEM((n_fields, BB), jnp.int32),
            pltpu.VMEM((n_idx,), jnp.int32),
            pltpu.VMEM((BB, PHYS_W), jnp.float32),
            pltpu.VMEM((1, EMBED_DIM, BB), jnp.float32),
            pltpu.SemaphoreType.DMA,
        ],
    )
    def body(xt_hbm, table_hbm, out_hbm, xblk, pidx, gath, ostg, sem):
        w = lax.axis_index("s") * NUM_CORES + lax.axis_index("c")
        b0 = pl.multiple_of(w * BB, PHYS_W)

        pltpu.sync_copy(xt_hbm.at[:, pl.ds(b0, BB)], xblk)

        def shift_body(i, carry):
            r = i // (BB // 16)
            cg = i % (BB // 16)
            v = xblk[r, pl.ds(cg * 16, 16)]
            pidx[pl.ds(r * BB + cg * 16, 16)] = v >> 2
            return carry

        lax.fori_loop(0, n_idx // 16, shift_body, 0)

        iota = lax.iota(jnp.int32, 16)

        def field_body(f, carry):
            pltpu.async_copy(
                table_hbm.at[pidx.at[pl.ds(f * BB, BB)]], gath, sem
            ).wait()

            def bg_body(bg, carry2):
                sv = xblk[f, pl.ds(bg * 16, 16)]
                scol = (sv & 3) * EMBED_DIM
                rows = bg * 16 + iota
                for k in range(EMBED_DIM):
                    vals = plsc.load_gather(gath, [rows, scol + k])
                    ostg[0, k, pl.ds(bg * 16, 16)] = vals
                return carry2

            lax.fori_loop(0, BB // 16, bg_body, 0)

            pltpu.sync_copy(
                ostg, out_hbm.at[pl.ds(f, 1), :, pl.ds(b0, BB)]
            )
            return carry

        lax.fori_loop(0, n_fields, field_body, 0)

    return body(xt, table_phys)


def kernel(x, table):
    batch, n_fields = x.shape
    xt = x.T.astype(jnp.int32)
    table_phys = table.reshape(table.shape[0] // FOLD, PHYS_W)
    out = _sc_embed(xt, table_phys, batch, n_fields)
    return out.transpose(2, 0, 1)


# speedup vs baseline: 1.6647x; 1.1581x over previous
import functools

import jax
import jax.numpy as jnp
from jax import lax
from jax.experimental import pallas as pl
from jax.experimental.pallas import tpu as pltpu
from jax.experimental.pallas import tpu_sc as plsc

EMBED_DIM = 32
PHYS_W = 128
FOLD = PHYS_W // EMBED_DIM

NUM_CORES = 2
NUM_SUBCORES = 16
NW = NUM_CORES * NUM_SUBCORES

BB = 512


def _sc_embed(xt, table_phys, batch, n_fields):
    mesh = plsc.VectorSubcoreMesh(core_axis_name="c", subcore_axis_name="s")
    n_idx = n_fields * BB

    @functools.partial(
        pl.kernel,
        mesh=mesh,
        out_type=jax.ShapeDtypeStruct((n_fields, EMBED_DIM, batch), jnp.float32),
        compiler_params=pltpu.CompilerParams(needs_layout_passes=False),
        scratch_types=[
            pltpu.VMEM((n_fields, BB), jnp.int32),
            pltpu.VMEM((n_idx,), jnp.int32),
            pltpu.VMEM((BB, PHYS_W), jnp.float32),
            pltpu.VMEM((1, EMBED_DIM, BB), jnp.float32),
            pltpu.SemaphoreType.DMA,
        ],
    )
    def body(xt_hbm, table_hbm, out_hbm, xblk, pidx, gath, ostg, sem):
        w = lax.axis_index("s") * NUM_CORES + lax.axis_index("c")
        b0 = pl.multiple_of(w * BB, PHYS_W)

        pltpu.sync_copy(xt_hbm.at[:, pl.ds(b0, BB)], xblk)

        def shift_body(i, carry):
            r = i // (BB // 16)
            cg = i % (BB // 16)
            v = xblk[r, pl.ds(cg * 16, 16)]
            pidx[pl.ds(r * BB + cg * 16, 16)] = v >> 2
            return carry

        lax.fori_loop(0, n_idx // 16, shift_body, 0)

        iota = lax.iota(jnp.int32, 16)

        def field_body(f, carry):
            pltpu.async_copy(
                table_hbm.at[pidx.at[pl.ds(f * BB, BB)]], gath, sem
            ).wait()

            def bg_body(bg, carry2):
                sv = xblk[f, pl.ds(bg * 16, 16)]
                scol = (sv & 3) * EMBED_DIM
                rows = bg * 16 + iota
                for kq in range(EMBED_DIM // 8):
                    vals = [
                        plsc.load_gather(gath, [rows, scol + (kq * 8 + j)])
                        for j in range(8)
                    ]
                    for j in range(8):
                        ostg[0, kq * 8 + j, pl.ds(bg * 16, 16)] = vals[j]
                return carry2

            lax.fori_loop(0, BB // 16, bg_body, 0)

            pltpu.sync_copy(
                ostg, out_hbm.at[pl.ds(f, 1), :, pl.ds(b0, BB)]
            )
            return carry

        lax.fori_loop(0, n_fields, field_body, 0)

    return body(xt, table_phys)


def kernel(x, table):
    batch, n_fields = x.shape
    xt = x.T.astype(jnp.int32)
    table_phys = table.reshape(table.shape[0] // FOLD, PHYS_W)
    out = _sc_embed(xt, table_phys, batch, n_fields)
    return out.transpose(2, 0, 1)


# speedup vs baseline: 1.8684x; 1.1223x over previous
import functools

import jax
import jax.numpy as jnp
from jax import lax
from jax.experimental import pallas as pl
from jax.experimental.pallas import tpu as pltpu
from jax.experimental.pallas import tpu_sc as plsc

EMBED_DIM = 32
PHYS_W = 128
FOLD = PHYS_W // EMBED_DIM

NUM_CORES = 2
NUM_SUBCORES = 16
NW = NUM_CORES * NUM_SUBCORES

BB = 512


def _sc_embed(xt, table_phys, batch, n_fields):
    mesh = plsc.VectorSubcoreMesh(core_axis_name="c", subcore_axis_name="s")
    n_idx = n_fields * BB

    @functools.partial(
        pl.kernel,
        mesh=mesh,
        out_type=jax.ShapeDtypeStruct((n_fields, EMBED_DIM, batch), jnp.float32),
        compiler_params=pltpu.CompilerParams(needs_layout_passes=False),
        scratch_types=[
            pltpu.VMEM((n_fields, BB), jnp.int32),
            pltpu.VMEM((n_idx,), jnp.int32),
            pltpu.VMEM((BB // 2, PHYS_W), jnp.float32),
            pltpu.VMEM((BB // 2, PHYS_W), jnp.float32),
            pltpu.VMEM((1, EMBED_DIM, BB), jnp.float32),
            pltpu.SemaphoreType.DMA,
            pltpu.SemaphoreType.DMA,
        ],
    )
    def body(xt_hbm, table_hbm, out_hbm, xblk, pidx, gathA, gathB, ostg,
             semA, semB):
        w = lax.axis_index("s") * NUM_CORES + lax.axis_index("c")
        b0 = pl.multiple_of(w * BB, PHYS_W)

        pltpu.sync_copy(xt_hbm.at[:, pl.ds(b0, BB)], xblk)

        def shift_body(i, carry):
            r = i // (BB // 16)
            cg = i % (BB // 16)
            v = xblk[r, pl.ds(cg * 16, 16)]
            pidx[pl.ds(r * BB + cg * 16, 16)] = v >> 2
            return carry

        lax.fori_loop(0, n_idx // 16, shift_body, 0)

        iota = lax.iota(jnp.int32, 16)
        HALF = BB // 2

        def fire(half_start, buf, sem):
            return pltpu.async_copy(
                table_hbm.at[pidx.at[pl.ds(half_start, HALF)]], buf, sem
            )

        def wait(buf, sem):
            pltpu.make_async_copy(
                table_hbm.at[pidx.at[pl.ds(0, HALF)]], buf, sem
            ).wait()

        def rearrange(buf, f, h):
            def bg_body(bg, carry2):
                sv = xblk[f, pl.ds(h * HALF + bg * 16, 16)]
                scol = (sv & 3) * EMBED_DIM
                rows = bg * 16 + iota
                for kq in range(EMBED_DIM // 8):
                    vals = [
                        plsc.load_gather(buf, [rows, scol + (kq * 8 + j)])
                        for j in range(8)
                    ]
                    for j in range(8):
                        ostg[0, kq * 8 + j,
                             pl.ds(h * HALF + bg * 16, 16)] = vals[j]
                return carry2

            lax.fori_loop(0, HALF // 16, bg_body, 0)

        fire(0, gathA, semA)

        def field_body(f, carry):
            fire(f * BB + HALF, gathB, semB)
            wait(gathA, semA)
            rearrange(gathA, f, 0)
            nxt = (f + 1) * BB
            nxt = jnp.where(nxt >= n_idx, 0, nxt)
            fire(nxt, gathA, semA)
            wait(gathB, semB)
            rearrange(gathB, f, 1)
            pltpu.sync_copy(ostg, out_hbm.at[pl.ds(f, 1), :, pl.ds(b0, BB)])
            return carry

        lax.fori_loop(0, n_fields, field_body, 0)
        wait(gathA, semA)

    return body(xt, table_phys)


def kernel(x, table):
    batch, n_fields = x.shape
    xt = x.T.astype(jnp.int32)
    table_phys = table.reshape(table.shape[0] // FOLD, PHYS_W)
    out = _sc_embed(xt, table_phys, batch, n_fields)
    return out.transpose(2, 0, 1)


# speedup vs baseline: 1.8796x; 1.0060x over previous
import functools

import jax
import jax.numpy as jnp
from jax import lax
from jax.experimental import pallas as pl
from jax.experimental.pallas import tpu as pltpu
from jax.experimental.pallas import tpu_sc as plsc

EMBED_DIM = 32
PHYS_W = 128
FOLD = PHYS_W // EMBED_DIM

NUM_CORES = 2
NUM_SUBCORES = 16
NW = NUM_CORES * NUM_SUBCORES

BB = 512


def _sc_embed(xt, table_phys, batch, n_fields):
    mesh = plsc.VectorSubcoreMesh(core_axis_name="c", subcore_axis_name="s")
    n_idx = n_fields * BB

    @functools.partial(
        pl.kernel,
        mesh=mesh,
        out_type=jax.ShapeDtypeStruct((n_fields, EMBED_DIM, batch), jnp.float32),
        compiler_params=pltpu.CompilerParams(
            needs_layout_passes=False, skip_device_barrier=True
        ),
        scratch_types=[
            pltpu.VMEM((n_fields, BB), jnp.int32),
            pltpu.VMEM((n_idx,), jnp.int32),
            pltpu.VMEM((BB // 2, PHYS_W), jnp.float32),
            pltpu.VMEM((BB // 2, PHYS_W), jnp.float32),
            pltpu.VMEM((1, EMBED_DIM, BB), jnp.float32),
            pltpu.SemaphoreType.DMA,
            pltpu.SemaphoreType.DMA,
            pltpu.SemaphoreType.DMA,
        ],
    )
    def body(xt_hbm, table_hbm, out_hbm, xblk, pidx, gathA, gathB, ostg,
             semA, semB, osem):
        w = lax.axis_index("s") * NUM_CORES + lax.axis_index("c")
        b0 = pl.multiple_of(w * BB, PHYS_W)

        pltpu.sync_copy(xt_hbm.at[:, pl.ds(b0, BB)], xblk)

        def shift_body(i, carry):
            r = i // (BB // 16)
            cg = i % (BB // 16)
            v = xblk[r, pl.ds(cg * 16, 16)]
            pidx[pl.ds(r * BB + cg * 16, 16)] = v >> 2
            return carry

        lax.fori_loop(0, n_idx // 16, shift_body, 0)

        iota = lax.iota(jnp.int32, 16)
        HALF = BB // 2

        def fire(half_start, buf, sem):
            return pltpu.async_copy(
                table_hbm.at[pidx.at[pl.ds(half_start, HALF)]], buf, sem
            )

        def wait(buf, sem):
            pltpu.make_async_copy(
                table_hbm.at[pidx.at[pl.ds(0, HALF)]], buf, sem
            ).wait()

        def rearrange(buf, f, h):
            def bg_body(bg, carry2):
                sv = xblk[f, pl.ds(h * HALF + bg * 16, 16)]
                scol = (sv & 3) * EMBED_DIM
                rows = bg * 16 + iota
                for kq in range(EMBED_DIM // 16):
                    vals = [
                        plsc.load_gather(buf, [rows, scol + (kq * 16 + j)])
                        for j in range(16)
                    ]
                    for j in range(16):
                        ostg[0, kq * 16 + j,
                             pl.ds(h * HALF + bg * 16, 16)] = vals[j]
                return carry2

            lax.fori_loop(0, HALF // 16, bg_body, 0)

        def out_slab(f):
            return out_hbm.at[pl.ds(f, 1), :, pl.ds(b0, BB)]

        fire(0, gathA, semA)
        pltpu.async_copy(ostg, out_slab(0), osem)

        def field_body(f, carry):
            fire(f * BB + HALF, gathB, semB)
            wait(gathA, semA)
            pltpu.make_async_copy(ostg, out_slab(f), osem).wait()
            rearrange(gathA, f, 0)
            nxt = (f + 1) * BB
            nxt = jnp.where(nxt >= n_idx, 0, nxt)
            fire(nxt, gathA, semA)
            wait(gathB, semB)
            rearrange(gathB, f, 1)
            pltpu.async_copy(ostg, out_slab(f), osem)
            return carry

        lax.fori_loop(0, n_fields, field_body, 0)
        wait(gathA, semA)
        pltpu.make_async_copy(ostg, out_slab(n_fields - 1), osem).wait()

    return body(xt, table_phys)


def kernel(x, table):
    batch, n_fields = x.shape
    xt = x.T.astype(jnp.int32)
    table_phys = table.reshape(table.shape[0] // FOLD, PHYS_W)
    out = _sc_embed(xt, table_phys, batch, n_fields)
    return out.transpose(2, 0, 1)
